# baseline (speedup 1.0000x reference)
import functools

import jax
import jax.numpy as jnp
from jax import lax
from jax.experimental import pallas as pl
from jax.experimental.pallas import tpu as pltpu
from jax.experimental.pallas import tpu_sc as plsc

_N = 10000
_E = 320000
_NC = 2
_NS = 16
_NW = _NC * _NS
_L = 16
_EPW = _E // _NW

_mesh = plsc.VectorSubcoreMesh(
    core_axis_name="c", subcore_axis_name="s", num_cores=_NC, num_subcores=_NS
)

_sc_params = pltpu.CompilerParams(needs_layout_passes=False)


def _worker_id():
    return lax.axis_index("c") * _NS + lax.axis_index("s")


_EW = (_E // (_NW * 128)) * 128
_XTRA = _E - _NW * _EW
_NX = _XTRA // 128
_NP = 10240
_SL = _NP // _NS


@functools.partial(
    pl.kernel,
    out_type=jax.ShapeDtypeStruct((_NW, _NP), jnp.float32),
    mesh=_mesh,
    scratch_types=[
        pltpu.VMEM((2, _EW), jnp.int32),
        pltpu.VMEM((2, 128), jnp.int32),
        pltpu.VMEM((_NP,), jnp.float32),
    ],
    compiler_params=_sc_params,
)
def _deg_partials(ei_hbm, out_hbm, win_v, xwin_v, hist_v):
    wid = _worker_id()
    zero16 = jnp.zeros((_L,), jnp.float32)

    @plsc.parallel_loop(0, _NP // _L, 1, unroll=8)
    def _zero(i):
        hist_v[pl.ds(i * _L, _L)] = zero16

    pltpu.sync_copy(ei_hbm.at[:, pl.ds(wid * _EW, _EW)], win_v)

    ones16 = jnp.ones((_L,), jnp.float32)

    @plsc.parallel_loop(0, _EW // _L, 1, unroll=8)
    def _scat(i):
        idx = win_v[1, pl.ds(i * _L, _L)]
        plsc.addupdate_scatter(hist_v, [idx], ones16)

    @pl.when(wid < _NX)
    def _extra():
        pltpu.sync_copy(ei_hbm.at[:, pl.ds(_NW * _EW + wid * 128, 128)], xwin_v)

        @plsc.parallel_loop(0, 128 // _L, 1, unroll=8)
        def _xscat(i):
            idx = xwin_v[1, pl.ds(i * _L, _L)]
            plsc.addupdate_scatter(hist_v, [idx], ones16)

    pltpu.sync_copy(hist_v, out_hbm.at[wid])


@functools.partial(
    pl.kernel,
    out_type=(
        jax.ShapeDtypeStruct((_NW, _NP), jnp.float32),
        jax.ShapeDtypeStruct((_NP,), jnp.float32),
    ),
    mesh=_mesh,
    scratch_types=[
        pltpu.VMEM((_NW, _SL), jnp.float32),
        pltpu.VMEM((_SL,), jnp.float32),
        pltpu.VMEM((_NP,), jnp.float32),
        pltpu.VMEM((2, _EW), jnp.int32),
        pltpu.VMEM((2, 128), jnp.int32),
        pltpu.VMEM((_NP,), jnp.float32),
        pltpu.VMEM_SHARED((_NP,), jnp.float32),
        pltpu.SemaphoreType.DMA,
        pltpu.SemaphoreType.DMA,
    ],
    compiler_params=_sc_params,
)
def _t_partials(ei_hbm, deg_hbm, out_hbm, dinv_out_hbm, part_v, slice_v,
                dinv_v, win_v, xwin_v, hist_v, dinv_sh, sem, esem):
    cid = lax.axis_index("c")
    sid = lax.axis_index("s")
    wid = cid * _NS + sid
    zero16 = jnp.zeros((_L,), jnp.float32)

    ewin = pltpu.async_copy(ei_hbm.at[:, pl.ds(wid * _EW, _EW)], win_v, esem)

    @plsc.parallel_loop(0, _NP // _L, 1, unroll=8)
    def _zero(i):
        hist_v[pl.ds(i * _L, _L)] = zero16

    copies = [
        pltpu.async_copy(deg_hbm.at[w, pl.ds(sid * _SL, _SL)], part_v.at[w], sem)
        for w in range(_NW)
    ]
    for c in copies:
        c.wait()

    half3 = jnp.full((_L,), 1.5, jnp.float32)
    magic = jnp.full((_L,), 0x5F3759DF, jnp.int32)

    @plsc.parallel_loop(0, _SL // _L, 1, unroll=2)
    def _dinv(j):
        acc = jnp.ones((_L,), jnp.float32)
        for w in range(_NW):
            acc = acc + part_v[w, pl.ds(j * _L, _L)]
        y = plsc.bitcast(
            magic - lax.shift_right_logical(plsc.bitcast(acc, jnp.int32), 1),
            jnp.float32)
        h = acc * 0.5
        y = y * (half3 - h * y * y)
        y = y * (half3 - h * y * y)
        y = y * (half3 - h * y * y)
        slice_v[pl.ds(j * _L, _L)] = y

    pltpu.sync_copy(slice_v, dinv_sh.at[pl.ds(sid * _SL, _SL)])
    plsc.subcore_barrier()
    pltpu.sync_copy(dinv_sh, dinv_v)

    @pl.when(cid == 0)
    def _emit_dinv():
        pltpu.sync_copy(slice_v, dinv_out_hbm.at[pl.ds(sid * _SL, _SL)])

    ewin.wait()

    @plsc.parallel_loop(0, _EW // _L, 1, unroll=8)
    def _edge(i):
        d = win_v[1, pl.ds(i * _L, _L)]
        srcs = win_v[0, pl.ds(i * _L, _L)]
        vals = plsc.load_gather(dinv_v, [d])
        plsc.addupdate_scatter(hist_v, [srcs], vals)

    @pl.when(wid < _NX)
    def _extra():
        pltpu.sync_copy(ei_hbm.at[:, pl.ds(_NW * _EW + wid * 128, 128)], xwin_v)

        @plsc.parallel_loop(0, 128 // _L, 1, unroll=8)
        def _xedge(i):
            d = xwin_v[1, pl.ds(i * _L, _L)]
            srcs = xwin_v[0, pl.ds(i * _L, _L)]
            vals = plsc.load_gather(dinv_v, [d])
            plsc.addupdate_scatter(hist_v, [srcs], vals)

    pltpu.sync_copy(hist_v, out_hbm.at[wid])


def _xw_body(f_ref, wg_ref, out_ref):
    out_ref[...] = jnp.dot(f_ref[...], wg_ref[...],
                           preferred_element_type=jnp.float32)


_xw_call = pl.pallas_call(
    _xw_body,
    out_shape=jax.ShapeDtypeStruct((_N, 64), jnp.float32),
)


def _final_body(tpart_ref, dinv_ref, xw_ref, bg_ref, out_ref):
    dinv = dinv_ref[...][None, :]
    t = jnp.sum(tpart_ref[...], axis=0, keepdims=True)
    coef = (dinv * (t + dinv))[:, :_N]
    o = jnp.dot(coef, xw_ref[...], preferred_element_type=jnp.float32)
    out_ref[...] = (o + _N * bg_ref[...]) * (1.0 / 16.0)


_final_call = pl.pallas_call(
    _final_body,
    out_shape=jax.ShapeDtypeStruct((1, 64), jnp.float32),
)


def kernel(features, edge_index, W_gcn, b_gcn, W1, b1, W2, b2):
    xw = _xw_call(features, W_gcn)
    deg_part = _deg_partials(edge_index)
    t_part, dinv = _t_partials(edge_index, deg_part)
    return _final_call(t_part, dinv, xw, b_gcn.reshape(1, -1))


# speedup vs baseline: 1.0299x; 1.0299x over previous
import functools

import jax
import jax.numpy as jnp
from jax import lax
from jax.experimental import pallas as pl
from jax.experimental.pallas import tpu as pltpu
from jax.experimental.pallas import tpu_sc as plsc

_N = 10000
_E = 320000
_NC = 2
_NS = 16
_NW = _NC * _NS
_L = 16
_EPW = _E // _NW

_mesh = plsc.VectorSubcoreMesh(
    core_axis_name="c", subcore_axis_name="s", num_cores=_NC, num_subcores=_NS
)

_sc_params = pltpu.CompilerParams(needs_layout_passes=False)


def _worker_id():
    return lax.axis_index("c") * _NS + lax.axis_index("s")


_EW = (_E // (_NW * 128)) * 128
_XTRA = _E - _NW * _EW
_NX = _XTRA // 128
_NP = 10240
_SL = _NP // _NS


@functools.partial(
    pl.kernel,
    out_type=jax.ShapeDtypeStruct((_NW, _NP), jnp.float32),
    mesh=_mesh,
    scratch_types=[
        pltpu.VMEM((2, _EW), jnp.int32),
        pltpu.VMEM((2, 128), jnp.int32),
        pltpu.VMEM((_NP,), jnp.float32),
    ],
    compiler_params=_sc_params,
)
def _deg_partials(ei_hbm, out_hbm, win_v, xwin_v, hist_v):
    wid = _worker_id()
    zero16 = jnp.zeros((_L,), jnp.float32)

    @plsc.parallel_loop(0, _NP // _L, 1, unroll=8)
    def _zero(i):
        hist_v[pl.ds(i * _L, _L)] = zero16

    pltpu.sync_copy(ei_hbm.at[:, pl.ds(wid * _EW, _EW)], win_v)

    ones16 = jnp.ones((_L,), jnp.float32)

    @plsc.parallel_loop(0, _EW // _L, 1, unroll=8)
    def _scat(i):
        idx = win_v[1, pl.ds(i * _L, _L)]
        plsc.addupdate_scatter(hist_v, [idx], ones16)

    @pl.when(wid < _NX)
    def _extra():
        pltpu.sync_copy(ei_hbm.at[:, pl.ds(_NW * _EW + wid * 128, 128)], xwin_v)

        @plsc.parallel_loop(0, 128 // _L, 1, unroll=8)
        def _xscat(i):
            idx = xwin_v[1, pl.ds(i * _L, _L)]
            plsc.addupdate_scatter(hist_v, [idx], ones16)

    pltpu.sync_copy(hist_v, out_hbm.at[wid])


@functools.partial(
    pl.kernel,
    out_type=(
        jax.ShapeDtypeStruct((_NW, _NP), jnp.float32),
        jax.ShapeDtypeStruct((_NP,), jnp.float32),
    ),
    mesh=_mesh,
    scratch_types=[
        pltpu.VMEM((_NW, _SL), jnp.float32),
        pltpu.VMEM((_SL,), jnp.float32),
        pltpu.VMEM((_NP,), jnp.float32),
        pltpu.VMEM((2, _EW), jnp.int32),
        pltpu.VMEM((2, 128), jnp.int32),
        pltpu.VMEM((_NP,), jnp.float32),
        pltpu.VMEM_SHARED((_NP,), jnp.float32),
        pltpu.SemaphoreType.DMA,
        pltpu.SemaphoreType.DMA,
    ],
    compiler_params=_sc_params,
)
def _t_partials(ei_hbm, deg_hbm, out_hbm, dinv_out_hbm, part_v, slice_v,
                dinv_v, win_v, xwin_v, hist_v, dinv_sh, sem, esem):
    cid = lax.axis_index("c")
    sid = lax.axis_index("s")
    wid = cid * _NS + sid
    zero16 = jnp.zeros((_L,), jnp.float32)

    ewin = pltpu.async_copy(ei_hbm.at[:, pl.ds(wid * _EW, _EW)], win_v, esem)

    @plsc.parallel_loop(0, _NP // _L, 1, unroll=8)
    def _zero(i):
        hist_v[pl.ds(i * _L, _L)] = zero16

    copies = [
        pltpu.async_copy(deg_hbm.at[w, pl.ds(sid * _SL, _SL)], part_v.at[w], sem)
        for w in range(_NW)
    ]
    for c in copies:
        c.wait()

    half3 = jnp.full((_L,), 1.5, jnp.float32)
    magic = jnp.full((_L,), 0x5F3759DF, jnp.int32)

    @plsc.parallel_loop(0, _SL // _L, 1, unroll=2)
    def _dinv(j):
        acc = jnp.ones((_L,), jnp.float32)
        for w in range(_NW):
            acc = acc + part_v[w, pl.ds(j * _L, _L)]
        y = plsc.bitcast(
            magic - lax.shift_right_logical(plsc.bitcast(acc, jnp.int32), 1),
            jnp.float32)
        h = acc * 0.5
        y = y * (half3 - h * y * y)
        y = y * (half3 - h * y * y)
        y = y * (half3 - h * y * y)
        slice_v[pl.ds(j * _L, _L)] = y

    pltpu.sync_copy(slice_v, dinv_sh.at[pl.ds(sid * _SL, _SL)])
    plsc.subcore_barrier()
    pltpu.sync_copy(dinv_sh, dinv_v)

    @pl.when(cid == 0)
    def _emit_dinv():
        pltpu.sync_copy(slice_v, dinv_out_hbm.at[pl.ds(sid * _SL, _SL)])

    ewin.wait()

    @plsc.parallel_loop(0, _EW // _L, 1, unroll=8)
    def _edge(i):
        d = win_v[1, pl.ds(i * _L, _L)]
        srcs = win_v[0, pl.ds(i * _L, _L)]
        vals = plsc.load_gather(dinv_v, [d])
        plsc.addupdate_scatter(hist_v, [srcs], vals)

    @pl.when(wid < _NX)
    def _extra():
        pltpu.sync_copy(ei_hbm.at[:, pl.ds(_NW * _EW + wid * 128, 128)], xwin_v)

        @plsc.parallel_loop(0, 128 // _L, 1, unroll=8)
        def _xedge(i):
            d = xwin_v[1, pl.ds(i * _L, _L)]
            srcs = xwin_v[0, pl.ds(i * _L, _L)]
            vals = plsc.load_gather(dinv_v, [d])
            plsc.addupdate_scatter(hist_v, [srcs], vals)

    pltpu.sync_copy(hist_v, out_hbm.at[wid])


def _final_body(tpart_ref, dinv_ref, f_ref, wg_ref, bg_ref, out_ref):
    dinv = dinv_ref[...][None, :]
    t = jnp.sum(tpart_ref[...], axis=0, keepdims=True)
    coef = (dinv * (t + dinv))[:, :_N]
    r = jnp.dot(coef, f_ref[...], preferred_element_type=jnp.float32)
    o = jnp.dot(r, wg_ref[...], preferred_element_type=jnp.float32)
    out_ref[...] = (o + _N * bg_ref[...]) * (1.0 / 16.0)


_final_call = pl.pallas_call(
    _final_body,
    out_shape=jax.ShapeDtypeStruct((1, 64), jnp.float32),
)


def kernel(features, edge_index, W_gcn, b_gcn, W1, b1, W2, b2):
    deg_part = _deg_partials(edge_index)
    t_part, dinv = _t_partials(edge_index, deg_part)
    return _final_call(t_part, dinv, features, W_gcn, b_gcn.reshape(1, -1))


# speedup vs baseline: 1.0359x; 1.0058x over previous
import functools

import jax
import jax.numpy as jnp
from jax import lax
from jax.experimental import pallas as pl
from jax.experimental.pallas import tpu as pltpu
from jax.experimental.pallas import tpu_sc as plsc

_N = 10000
_E = 320000
_NC = 2
_NS = 16
_NW = _NC * _NS
_L = 16
_EPW = _E // _NW

_mesh = plsc.VectorSubcoreMesh(
    core_axis_name="c", subcore_axis_name="s", num_cores=_NC, num_subcores=_NS
)

_sc_params = pltpu.CompilerParams(needs_layout_passes=False)


def _worker_id():
    return lax.axis_index("c") * _NS + lax.axis_index("s")


_EW = (_E // (_NW * 128)) * 128
_XTRA = _E - _NW * _EW
_NX = _XTRA // 128
_NP = 10240
_SL = _NP // _NS


@functools.partial(
    pl.kernel,
    out_type=jax.ShapeDtypeStruct((_NW, _NP), jnp.float32),
    mesh=_mesh,
    scratch_types=[
        pltpu.VMEM((2, _EW), jnp.int32),
        pltpu.VMEM((2, 128), jnp.int32),
        pltpu.VMEM((_NP,), jnp.float32),
        pltpu.SemaphoreType.DMA,
    ],
    compiler_params=_sc_params,
)
def _deg_partials(ei_hbm, out_hbm, win_v, xwin_v, hist_v, wsem):
    wid = _worker_id()
    zero16 = jnp.zeros((_L,), jnp.float32)

    ewin = pltpu.async_copy(ei_hbm.at[:, pl.ds(wid * _EW, _EW)], win_v, wsem)

    @plsc.parallel_loop(0, _NP // _L, 1, unroll=8)
    def _zero(i):
        hist_v[pl.ds(i * _L, _L)] = zero16

    ewin.wait()

    ones16 = jnp.ones((_L,), jnp.float32)

    @plsc.parallel_loop(0, _EW // _L, 1, unroll=8)
    def _scat(i):
        idx = win_v[1, pl.ds(i * _L, _L)]
        plsc.addupdate_scatter(hist_v, [idx], ones16)

    @pl.when(wid < _NX)
    def _extra():
        pltpu.sync_copy(ei_hbm.at[:, pl.ds(_NW * _EW + wid * 128, 128)], xwin_v)

        @plsc.parallel_loop(0, 128 // _L, 1, unroll=8)
        def _xscat(i):
            idx = xwin_v[1, pl.ds(i * _L, _L)]
            plsc.addupdate_scatter(hist_v, [idx], ones16)

    pltpu.sync_copy(hist_v, out_hbm.at[wid])


@functools.partial(
    pl.kernel,
    out_type=(
        jax.ShapeDtypeStruct((_NW, _NP), jnp.float32),
        jax.ShapeDtypeStruct((_NP,), jnp.float32),
    ),
    mesh=_mesh,
    scratch_types=[
        pltpu.VMEM((_NW, _SL), jnp.float32),
        pltpu.VMEM((_SL,), jnp.float32),
        pltpu.VMEM((_NP,), jnp.float32),
        pltpu.VMEM((2, _EW), jnp.int32),
        pltpu.VMEM((2, 128), jnp.int32),
        pltpu.VMEM((_NP,), jnp.float32),
        pltpu.VMEM_SHARED((_NP,), jnp.float32),
        pltpu.SemaphoreType.DMA,
        pltpu.SemaphoreType.DMA,
    ],
    compiler_params=_sc_params,
)
def _t_partials(ei_hbm, deg_hbm, out_hbm, dinv_out_hbm, part_v, slice_v,
                dinv_v, win_v, xwin_v, hist_v, dinv_sh, sem, esem):
    cid = lax.axis_index("c")
    sid = lax.axis_index("s")
    wid = cid * _NS + sid
    zero16 = jnp.zeros((_L,), jnp.float32)

    ewin = pltpu.async_copy(ei_hbm.at[:, pl.ds(wid * _EW, _EW)], win_v, esem)

    copies = [
        pltpu.async_copy(deg_hbm.at[w, pl.ds(sid * _SL, _SL)], part_v.at[w], sem)
        for w in range(_NW)
    ]

    @plsc.parallel_loop(0, _NP // _L, 1, unroll=8)
    def _zero(i):
        hist_v[pl.ds(i * _L, _L)] = zero16

    for c in copies:
        c.wait()

    half3 = jnp.full((_L,), 1.5, jnp.float32)
    magic = jnp.full((_L,), 0x5F3759DF, jnp.int32)

    @plsc.parallel_loop(0, _SL // _L, 1, unroll=2)
    def _dinv(j):
        acc = jnp.ones((_L,), jnp.float32)
        for w in range(_NW):
            acc = acc + part_v[w, pl.ds(j * _L, _L)]
        y = plsc.bitcast(
            magic - lax.shift_right_logical(plsc.bitcast(acc, jnp.int32), 1),
            jnp.float32)
        h = acc * 0.5
        y = y * (half3 - h * y * y)
        y = y * (half3 - h * y * y)
        y = y * (half3 - h * y * y)
        slice_v[pl.ds(j * _L, _L)] = y

    pltpu.sync_copy(slice_v, dinv_sh.at[pl.ds(sid * _SL, _SL)])
    plsc.subcore_barrier()
    pltpu.sync_copy(dinv_sh, dinv_v)

    @pl.when(cid == 0)
    def _emit_dinv():
        pltpu.sync_copy(slice_v, dinv_out_hbm.at[pl.ds(sid * _SL, _SL)])

    ewin.wait()

    @plsc.parallel_loop(0, _EW // _L, 1, unroll=8)
    def _edge(i):
        d = win_v[1, pl.ds(i * _L, _L)]
        srcs = win_v[0, pl.ds(i * _L, _L)]
        vals = plsc.load_gather(dinv_v, [d])
        plsc.addupdate_scatter(hist_v, [srcs], vals)

    @pl.when(wid < _NX)
    def _extra():
        pltpu.sync_copy(ei_hbm.at[:, pl.ds(_NW * _EW + wid * 128, 128)], xwin_v)

        @plsc.parallel_loop(0, 128 // _L, 1, unroll=8)
        def _xedge(i):
            d = xwin_v[1, pl.ds(i * _L, _L)]
            srcs = xwin_v[0, pl.ds(i * _L, _L)]
            vals = plsc.load_gather(dinv_v, [d])
            plsc.addupdate_scatter(hist_v, [srcs], vals)

    pltpu.sync_copy(hist_v, out_hbm.at[wid])


def _final_body(tpart_ref, dinv_ref, f_ref, wg_ref, bg_ref, out_ref):
    dinv = dinv_ref[...][None, :]
    t = jnp.sum(tpart_ref[...], axis=0, keepdims=True)
    coef = (dinv * (t + dinv))[:, :_N]
    r = jnp.dot(coef, f_ref[...], preferred_element_type=jnp.float32)
    o = jnp.dot(r, wg_ref[...], preferred_element_type=jnp.float32)
    out_ref[...] = (o + _N * bg_ref[...]) * (1.0 / 16.0)


_final_call = pl.pallas_call(
    _final_body,
    out_shape=jax.ShapeDtypeStruct((1, 64), jnp.float32),
)


def kernel(features, edge_index, W_gcn, b_gcn, W1, b1, W2, b2):
    deg_part = _deg_partials(edge_index)
    t_part, dinv = _t_partials(edge_index, deg_part)
    return _final_call(t_part, dinv, features, W_gcn, b_gcn.reshape(1, -1))


# speedup vs baseline: 1.0926x; 1.0547x over previous
import functools

import jax
import jax.numpy as jnp
from jax import lax
from jax.experimental import pallas as pl
from jax.experimental.pallas import tpu as pltpu
from jax.experimental.pallas import tpu_sc as plsc

_N = 10000
_E = 320000
_NC = 2
_NS = 16
_NW = _NC * _NS
_L = 16

_EW = (_E // (_NW * 128)) * 128
_XTRA = _E - _NW * _EW
_NX = _XTRA // 128

_mesh = plsc.VectorSubcoreMesh(
    core_axis_name="c", subcore_axis_name="s", num_cores=_NC, num_subcores=_NS
)

_sc_params = pltpu.CompilerParams(needs_layout_passes=False)


def _worker_id():
    return lax.axis_index("c") * _NS + lax.axis_index("s")


@functools.partial(
    pl.kernel,
    out_type=jax.ShapeDtypeStruct((_NW, _N), jnp.float32),
    mesh=_mesh,
    scratch_types=[
        pltpu.VMEM((2, _EW), jnp.int32),
        pltpu.VMEM((2, 128), jnp.int32),
        pltpu.VMEM((_N,), jnp.float32),
        pltpu.SemaphoreType.DMA,
    ],
    compiler_params=_sc_params,
)
def _deg_partials(ei_hbm, out_hbm, win_v, xwin_v, hist_v, wsem):
    wid = _worker_id()
    zero16 = jnp.zeros((_L,), jnp.float32)

    ewin = pltpu.async_copy(ei_hbm.at[:, pl.ds(wid * _EW, _EW)], win_v, wsem)

    @plsc.parallel_loop(0, _N // _L, 1, unroll=8)
    def _zero(i):
        hist_v[pl.ds(i * _L, _L)] = zero16

    ewin.wait()

    ones16 = jnp.ones((_L,), jnp.float32)

    @plsc.parallel_loop(0, _EW // _L, 1, unroll=8)
    def _scat(i):
        idx = win_v[1, pl.ds(i * _L, _L)]
        plsc.addupdate_scatter(hist_v, [idx], ones16)

    @pl.when(wid < _NX)
    def _extra():
        pltpu.sync_copy(ei_hbm.at[:, pl.ds(_NW * _EW + wid * 128, 128)], xwin_v)

        @plsc.parallel_loop(0, 128 // _L, 1, unroll=8)
        def _xscat(i):
            idx = xwin_v[1, pl.ds(i * _L, _L)]
            plsc.addupdate_scatter(hist_v, [idx], ones16)

    pltpu.sync_copy(hist_v, out_hbm.at[wid])


@functools.partial(
    pl.kernel,
    out_type=jax.ShapeDtypeStruct((_NW, _N), jnp.float32),
    mesh=_mesh,
    scratch_types=[
        pltpu.VMEM((_N,), jnp.float32),
        pltpu.VMEM((2, _EW), jnp.int32),
        pltpu.VMEM((2, 128), jnp.int32),
        pltpu.VMEM((_N,), jnp.float32),
        pltpu.SemaphoreType.DMA,
    ],
    compiler_params=_sc_params,
)
def _t_partials(ei_hbm, dinv_hbm, out_hbm, dinv_v, win_v, xwin_v, hist_v, wsem):
    wid = _worker_id()
    zero16 = jnp.zeros((_L,), jnp.float32)

    ewin = pltpu.async_copy(ei_hbm.at[:, pl.ds(wid * _EW, _EW)], win_v, wsem)

    @plsc.parallel_loop(0, _N // _L, 1, unroll=8)
    def _zero(i):
        hist_v[pl.ds(i * _L, _L)] = zero16

    pltpu.sync_copy(dinv_hbm, dinv_v)
    ewin.wait()

    @plsc.parallel_loop(0, _EW // _L, 1, unroll=8)
    def _edge(i):
        d = win_v[1, pl.ds(i * _L, _L)]
        srcs = win_v[0, pl.ds(i * _L, _L)]
        vals = plsc.load_gather(dinv_v, [d])
        plsc.addupdate_scatter(hist_v, [srcs], vals)

    @pl.when(wid < _NX)
    def _extra():
        pltpu.sync_copy(ei_hbm.at[:, pl.ds(_NW * _EW + wid * 128, 128)], xwin_v)

        @plsc.parallel_loop(0, 128 // _L, 1, unroll=8)
        def _xedge(i):
            d = xwin_v[1, pl.ds(i * _L, _L)]
            srcs = xwin_v[0, pl.ds(i * _L, _L)]
            vals = plsc.load_gather(dinv_v, [d])
            plsc.addupdate_scatter(hist_v, [srcs], vals)

    pltpu.sync_copy(hist_v, out_hbm.at[wid])


def _dinv_body(part_ref, out_ref):
    deg = jnp.sum(part_ref[...], axis=0) + 1.0
    out_ref[...] = lax.rsqrt(deg)


_dinv_call = pl.pallas_call(
    _dinv_body,
    out_shape=jax.ShapeDtypeStruct((_N,), jnp.float32),
)


def _final_body(tpart_ref, dinv_ref, f_ref, wg_ref, bg_ref, out_ref):
    dinv = dinv_ref[...][None, :]
    t = jnp.sum(tpart_ref[...], axis=0, keepdims=True)
    coef = dinv * (t + dinv)
    r = jnp.dot(coef, f_ref[...], preferred_element_type=jnp.float32)
    o = jnp.dot(r, wg_ref[...], preferred_element_type=jnp.float32)
    out_ref[...] = (o + _N * bg_ref[...]) * (1.0 / 16.0)


_final_call = pl.pallas_call(
    _final_body,
    out_shape=jax.ShapeDtypeStruct((1, 64), jnp.float32),
)


def kernel(features, edge_index, W_gcn, b_gcn, W1, b1, W2, b2):
    deg_part = _deg_partials(edge_index)
    dinv = _dinv_call(deg_part)
    t_part = _t_partials(edge_index, dinv)
    return _final_call(t_part, dinv, features, W_gcn, b_gcn.reshape(1, -1))
